# baseline (speedup 1.0000x reference)
import functools

import jax
import jax.numpy as jnp
from jax import lax
from jax.experimental import pallas as pl
from jax.experimental.pallas import tpu as pltpu
from jax.experimental.pallas import tpu_sc as plsc

BATCH = 16384
DIM = 32
NC = 2
NS = 16
L = 16
NW = NC * NS
BPW = BATCH // NW
CHUNK = 128
NCHUNK = BPW // CHUNK


def _sc_body(didx_hbm, aidx_hbm, dtab_hbm, atab_hbm, out_hbm,
             didx_v, aidx_v, drows_v, arows_v, out_v, sem):
    wid = lax.axis_index("s") * NC + lax.axis_index("c")

    pltpu.sync_copy(didx_hbm.at[pl.ds(wid * NCHUNK, NCHUNK)], didx_v)
    pltpu.sync_copy(aidx_hbm.at[pl.ds(wid * NCHUNK, NCHUNK)], aidx_v)

    drows2 = drows_v
    arows2 = arows_v
    copies = []
    for c in range(NCHUNK):
        copies.append(pltpu.async_copy(
            dtab_hbm.at[didx_v.at[c]], drows2.at[pl.ds(c * CHUNK, CHUNK)],
            sem))
        copies.append(pltpu.async_copy(
            atab_hbm.at[aidx_v.at[c]], arows2.at[pl.ds(c * CHUNK, CHUNK)],
            sem))
    for cp in copies:
        cp.wait()

    lane = lax.iota(jnp.int32, L)

    def group(g, carry):
        rows = g * L + lane
        acc = jnp.zeros((L,), jnp.float32)
        for j in range(DIM):
            col = jnp.full((L,), j, jnp.int32)
            dv = plsc.load_gather(drows_v, [rows, col])
            av = plsc.load_gather(arows_v, [rows, col])
            acc = acc + dv * av
        out_v[pl.ds(g * L, L)] = acc
        return carry

    lax.fori_loop(0, BPW // L, group, 0)

    pltpu.sync_copy(out_v, out_hbm.at[pl.ds(wid * BPW, BPW)])


@functools.partial(
    pl.kernel,
    mesh=plsc.VectorSubcoreMesh(core_axis_name="c", subcore_axis_name="s"),
    out_type=jax.ShapeDtypeStruct((BATCH,), jnp.float32),
    scratch_types=[
        pltpu.VMEM((NCHUNK, CHUNK), jnp.int32),
        pltpu.VMEM((NCHUNK, CHUNK), jnp.int32),
        pltpu.VMEM((BPW, DIM), jnp.float32),
        pltpu.VMEM((BPW, DIM), jnp.float32),
        pltpu.VMEM((BPW,), jnp.float32),
        pltpu.SemaphoreType.DMA,
    ],
    compiler_params=pltpu.CompilerParams(
        needs_layout_passes=False, use_tc_tiling_on_sc=False),
)
def _sc_call(didx_hbm, aidx_hbm, dtab_hbm, atab_hbm, out_hbm,
             didx_v, aidx_v, drows_v, arows_v, out_v, sem):
    _sc_body(didx_hbm, aidx_hbm, dtab_hbm, atab_hbm, out_hbm,
             didx_v, aidx_v, drows_v, arows_v, out_v, sem)


@jax.jit
def kernel(drug_idx, adr_idx, drug_embeddings, adr_embeddings):
    didx2 = drug_idx.reshape(NW * NCHUNK, CHUNK)
    aidx2 = adr_idx.reshape(NW * NCHUNK, CHUNK)
    return _sc_call(didx2, aidx2, drug_embeddings, adr_embeddings)
